# Initial kernel scaffold; baseline (speedup 1.0000x reference)
import jax
import jax.numpy as jnp
from jax.experimental import pallas as pl


def kernel(pred_cls_logits, pred_box_deltas, proposal_boxes):
    raise NotImplementedError("write your pallas kernel here")


# speedup vs baseline: 1.7829x; 1.7829x over previous
import jax
import jax.numpy as jnp
from jax.experimental import pallas as pl
from jax.experimental.pallas import tpu as pltpu

_C = 18
_K = 1024
_SCORE_T = 0.05
_NMS_T = 0.25


def _nms_kernel(sc_col_ref, sc_row_ref, cls_col_ref, cls_row_ref,
                ctr_ref, off_ref, dlt_ref,
                ctrT_ref, offT_ref, dltT_ref,
                out_ref, over_ref):
    scores_c = sc_col_ref[0]
    scores_r = sc_row_ref[0]
    cls_c = cls_col_ref[0]
    cls_r = cls_row_ref[0]
    ctr = ctr_ref[0]
    off = off_ref[0]
    dlt = dlt_ref[0]
    ctrT = ctrT_ref[0]
    offT = offT_ref[0]
    dltT = dltT_ref[0]

    po = off + dlt
    mins = ctr - po[:, 0:3]
    maxs = ctr + po[:, 3:6]
    poT = offT + dltT
    minsT = ctrT - poT[0:3, :]
    maxsT = ctrT + poT[3:6, :]

    mc = jnp.maximum(jnp.max(jnp.abs(mins)), jnp.max(jnp.abs(maxs))) + 1.0
    sh_c = cls_c * (2.0 * mc)
    sh_r = cls_r * (2.0 * mc)
    smin = mins + sh_c
    smax = maxs + sh_c
    sminT = minsT + sh_r
    smaxT = maxsT + sh_r

    d_c = jnp.maximum(smax - smin, 0.0)
    vol_c = d_c[:, 0:1] * d_c[:, 1:2] * d_c[:, 2:3]
    d_r = jnp.maximum(smaxT - sminT, 0.0)
    vol_r = d_r[0:1, :] * d_r[1:2, :] * d_r[2:3, :]

    inter = None
    for d in range(3):
        imin = jnp.maximum(smin[:, d:d + 1], sminT[d:d + 1, :])
        imax = jnp.minimum(smax[:, d:d + 1], smaxT[d:d + 1, :])
        w = jnp.maximum(imax - imin, 0.0)
        inter = w if inter is None else inter * w
    union = jnp.maximum(vol_c + vol_r - inter, 1e-8)
    over_ref[...] = ((inter / union) > _NMS_T).astype(jnp.float32)

    idxs = jax.lax.broadcasted_iota(jnp.int32, (1, _K), 1)
    keep0 = (scores_r > _SCORE_T).astype(jnp.float32)

    def body(i, keep):
        row = over_ref[pl.ds(i, 1), :]
        lt = (idxs < i).astype(jnp.float32)
        sup = jnp.max(row * keep * lt)
        supf = (sup > 0.0).astype(jnp.float32)
        return jnp.where(idxs == i, keep * (1.0 - supf), keep)

    keep = jax.lax.fori_loop(1, _K, body, keep0)

    r = jax.lax.broadcasted_iota(jnp.int32, (_K, _K), 0)
    c = jax.lax.broadcasted_iota(jnp.int32, (_K, _K), 1)
    eye = (r == c).astype(jnp.float32)
    keep_col = jax.lax.dot_general(
        eye, keep, (((1,), (1,)), ((), ())),
        preferred_element_type=jnp.float32)

    out = jnp.concatenate([mins, maxs, scores_c], axis=1)
    out_ref[0] = jnp.where(keep_col > 0.0, out, 0.0)


def kernel(pred_cls_logits, pred_box_deltas, proposal_boxes):
    B, C1, N = pred_cls_logits.shape
    C = C1 - 1

    scores = jax.nn.softmax(pred_cls_logits, axis=1)
    scores = jnp.transpose(scores, (0, 2, 1))[:, :, :C]
    flat = jnp.where(scores > _SCORE_T, scores, 0.0).reshape(B, N * C)
    top_scores, top_idx = jax.lax.top_k(flat, _K)
    n_idx = top_idx // C
    c_idx = top_idx % C

    dlt = jnp.take_along_axis(
        pred_box_deltas.reshape(B, N * C, 6), top_idx[:, :, None], axis=1)
    props = jnp.take_along_axis(proposal_boxes, n_idx[:, :, None], axis=1)
    ctr = props[:, :, 0:3]
    off = props[:, :, 3:9]
    clsf = c_idx.astype(jnp.float32)

    sc_col = top_scores[:, :, None]
    sc_row = top_scores[:, None, :]
    cls_col = clsf[:, :, None]
    cls_row = clsf[:, None, :]
    ctrT = jnp.transpose(ctr, (0, 2, 1))
    offT = jnp.transpose(off, (0, 2, 1))
    dltT = jnp.transpose(dlt, (0, 2, 1))

    def spec(s1, s2):
        return pl.BlockSpec((1, s1, s2), lambda b: (b, 0, 0))

    out = pl.pallas_call(
        _nms_kernel,
        grid=(B,),
        in_specs=[spec(_K, 1), spec(1, _K), spec(_K, 1), spec(1, _K),
                  spec(_K, 3), spec(_K, 6), spec(_K, 6),
                  spec(3, _K), spec(6, _K), spec(6, _K)],
        out_specs=spec(_K, 7),
        out_shape=jax.ShapeDtypeStruct((B, _K, 7), jnp.float32),
        scratch_shapes=[pltpu.VMEM((_K, _K), jnp.float32)],
    )(sc_col, sc_row, cls_col, cls_row, ctr, off, dlt, ctrT, offT, dltT)
    return out


# speedup vs baseline: 2.4102x; 1.3518x over previous
import jax
import jax.numpy as jnp
from jax.experimental import pallas as pl
from jax.experimental.pallas import tpu as pltpu

_C = 18
_K = 1024
_SCORE_T = 0.05
_NMS_T = 0.25


def _nms_kernel(colpack_ref, rowpack_ref, out_ref, over_ref):
    B = colpack_ref.shape[0]

    minsT_all = []
    maxsT_all = []
    for b in range(B):
        cp = colpack_ref[b]
        rp = rowpack_ref[b]
        cls_c = cp[:, 1:2]
        ctr = cp[:, 2:5]
        off = cp[:, 5:11]
        dlt = cp[:, 11:17]
        cls_r = rp[1:2, :]
        ctrT = rp[2:5, :]
        offT = rp[5:11, :]
        dltT = rp[11:17, :]

        po = off + dlt
        mins = ctr - po[:, 0:3]
        maxs = ctr + po[:, 3:6]
        poT = offT + dltT
        minsT = ctrT - poT[0:3, :]
        maxsT = ctrT + poT[3:6, :]
        minsT_all.append(minsT)
        maxsT_all.append(maxsT)

        mc = jnp.maximum(jnp.max(jnp.abs(mins)), jnp.max(jnp.abs(maxs))) + 1.0
        sh_c = cls_c * (2.0 * mc)
        sh_r = cls_r * (2.0 * mc)
        smin = mins + sh_c
        smax = maxs + sh_c
        sminT = minsT + sh_r
        smaxT = maxsT + sh_r

        d_c = jnp.maximum(smax - smin, 0.0)
        vol_c = d_c[:, 0:1] * d_c[:, 1:2] * d_c[:, 2:3]
        d_r = jnp.maximum(smaxT - sminT, 0.0)
        vol_r = d_r[0:1, :] * d_r[1:2, :] * d_r[2:3, :]

        _CH = 128
        for g in range(_K // _CH):
            lo = g * _CH
            hi = lo + _CH
            inter = None
            for d in range(3):
                imin = jnp.maximum(smin[lo:hi, d:d + 1], sminT[d:d + 1, :])
                imax = jnp.minimum(smax[lo:hi, d:d + 1], smaxT[d:d + 1, :])
                w = jnp.maximum(imax - imin, 0.0)
                inter = w if inter is None else inter * w
            union = jnp.maximum(vol_c[lo:hi] + vol_r - inter, 1e-8)
            over_ref[lo:hi, b, :] = ((inter / union) > _NMS_T).astype(
                jnp.float32)

    idxs = jax.lax.broadcasted_iota(jnp.int32, (1, _K), 1)
    keep0 = (rowpack_ref[:, 0, :] > _SCORE_T).astype(jnp.float32)

    def body(i, keep):
        row = over_ref[i]
        lt = (idxs < i).astype(jnp.float32)
        sup = jnp.max(row * keep * lt, axis=1, keepdims=True)
        supf = (sup > 0.0).astype(jnp.float32)
        eq = (idxs == i).astype(jnp.float32)
        return keep * (1.0 - supf * eq)

    keep = jax.lax.fori_loop(1, _K, body, keep0)

    for b in range(B):
        outT = jnp.concatenate(
            [minsT_all[b], maxsT_all[b], rowpack_ref[b][0:1, :]], axis=0)
        out_ref[b] = jnp.where(keep[b:b + 1, :] > 0.0, outT, 0.0)


def kernel(pred_cls_logits, pred_box_deltas, proposal_boxes):
    B, C1, N = pred_cls_logits.shape
    C = C1 - 1

    scores = jax.nn.softmax(pred_cls_logits, axis=1)
    scores = jnp.transpose(scores, (0, 2, 1))[:, :, :C]
    flat = jnp.where(scores > _SCORE_T, scores, 0.0).reshape(B, N * C)
    top_scores, top_idx = jax.lax.top_k(flat, _K)
    n_idx = top_idx // C
    c_idx = top_idx % C

    dlt = jnp.take_along_axis(
        pred_box_deltas.reshape(B, N * C, 6), top_idx[:, :, None], axis=1)
    props = jnp.take_along_axis(proposal_boxes, n_idx[:, :, None], axis=1)
    clsf = c_idx.astype(jnp.float32)

    colpack = jnp.concatenate(
        [top_scores[:, :, None], clsf[:, :, None], props, dlt], axis=2)
    rowpack = jnp.transpose(colpack, (0, 2, 1))

    outT = pl.pallas_call(
        _nms_kernel,
        out_shape=jax.ShapeDtypeStruct((B, 7, _K), jnp.float32),
        scratch_shapes=[pltpu.VMEM((_K, B, _K), jnp.float32)],
    )(colpack, rowpack)
    return jnp.transpose(outT, (0, 2, 1))
